# Initial kernel scaffold; baseline (speedup 1.0000x reference)
import jax
import jax.numpy as jnp
from jax.experimental import pallas as pl


def kernel(x, neighbor_indices, distancesq, Wd0, bd0, Wf0, bf0, Wd1, bd1, Wf1, bf1, Wd2, bd2, Wf2, bf2):
    raise NotImplementedError("write your pallas kernel here")


# speedup vs baseline: 1.3392x; 1.3392x over previous
import functools

import jax
import jax.numpy as jnp
from jax import lax
from jax.experimental import pallas as pl
from jax.experimental.pallas import tpu as pltpu
from jax.experimental.pallas import tpu_sc as plsc

V = 10000
D = 128
K = 32
F = 64

NC = 2
NS = 16
L = 16
NW = NC * NS
VP = 10240
RPW = VP // NW
CH = 4
GB = CH * K
NCHUNK = RPW // CH


def _tc_body(x_ref, f_ref, d_ref, wdx_ref, wdf_ref, bd_ref, wf_ref, bf_ref,
             w_ref, feat_ref, dn_ref):
    xb = x_ref[...]
    fb = f_ref[...]
    s = (jnp.sum(xb * wdx_ref[...], axis=1, keepdims=True)
         + jnp.sum(fb * wdf_ref[...], axis=1, keepdims=True)
         + bd_ref[0, 0])
    scale = 10.0 / (1.0 + jnp.exp(-s))
    dn = d_ref[...] * scale
    dn_ref[...] = dn
    w_ref[...] = jnp.exp(-10.0 * dn)
    ft = jnp.maximum(
        jnp.dot(fb, wf_ref[...], preferred_element_type=jnp.float32)
        + bf_ref[...], 0.0)
    feat_ref[...] = jnp.concatenate(
        [ft, jnp.zeros_like(ft)], axis=1)


def _tc_stage(x, featin, d, wdx, wdf, bd, wf, bf):
    bv = 1000
    grid = (V // bv,)
    return pl.pallas_call(
        _tc_body,
        grid=grid,
        in_specs=[
            pl.BlockSpec((bv, D), lambda i: (i, 0)),
            pl.BlockSpec((bv, D), lambda i: (i, 0)),
            pl.BlockSpec((bv, K), lambda i: (i, 0)),
            pl.BlockSpec((1, D), lambda i: (0, 0)),
            pl.BlockSpec((1, D), lambda i: (0, 0)),
            pl.BlockSpec((1, 1), lambda i: (0, 0)),
            pl.BlockSpec((D, F), lambda i: (0, 0)),
            pl.BlockSpec((1, F), lambda i: (0, 0)),
        ],
        out_specs=[
            pl.BlockSpec((bv, K), lambda i: (i, 0)),
            pl.BlockSpec((bv, 2 * F), lambda i: (i, 0)),
            pl.BlockSpec((bv, K), lambda i: (i, 0)),
        ],
        out_shape=[
            jax.ShapeDtypeStruct((V, K), jnp.float32),
            jax.ShapeDtypeStruct((V, 2 * F), jnp.float32),
            jax.ShapeDtypeStruct((V, K), jnp.float32),
        ],
    )(x, featin, d, wdx, wdf, bd, wf, bf)


def _sc_body(feat_hbm, nidxf_hbm, wflat_hbm, out_hbm,
             idx_v, rows_v, w_v, own_v, out_v, sem):
    wid = lax.axis_index("s") * NC + lax.axis_index("c")
    base = wid * RPW

    def chunk(c, carry):
        r0 = base + c * CH
        pltpu.sync_copy(nidxf_hbm.at[pl.ds(r0 * K, GB)], idx_v)
        pltpu.sync_copy(wflat_hbm.at[pl.ds(r0 * K, GB)], w_v)
        pltpu.sync_copy(feat_hbm.at[pl.ds(r0, CH)], own_v)
        pltpu.async_copy(feat_hbm.at[idx_v], rows_v, sem).wait()

        for dl in range(CH):
            row0 = dl * K
            accs = [jnp.zeros((L,), jnp.float32) for _ in range(F // L)]
            accm = [jnp.full((L,), -jnp.inf, jnp.float32)
                    for _ in range(F // L)]
            for kg in range(K // L):
                w16 = w_v[pl.ds(row0 + kg * L, L)]
                for kl in range(L):
                    k = kg * L + kl
                    wv = jnp.full((L,), w16[kl])
                    for t in range(F // L):
                        nf = rows_v[row0 + k, pl.ds(t * L, L)]
                        wfv = wv * nf
                        accs[t] = accs[t] + wfv
                        accm[t] = jnp.maximum(accm[t], wfv)
            for t in range(F // L):
                ov = own_v[dl, pl.ds(t * L, L)]
                out_v[dl, pl.ds(t * L, L)] = accs[t] * (1.0 / K) - ov
                out_v[dl, pl.ds(F + t * L, L)] = accm[t] - ov
        pltpu.sync_copy(out_v, out_hbm.at[pl.ds(r0, CH)])
        return carry

    lax.fori_loop(0, NCHUNK, chunk, 0)


_sc_knn = functools.partial(
    pl.kernel,
    out_type=jax.ShapeDtypeStruct((VP, 2 * F), jnp.float32),
    mesh=plsc.VectorSubcoreMesh(
        core_axis_name="c", subcore_axis_name="s",
        num_cores=NC, num_subcores=NS),
    scratch_types=[
        pltpu.VMEM((GB,), jnp.int32),
        pltpu.VMEM((GB, 2 * F), jnp.float32),
        pltpu.VMEM((GB,), jnp.float32),
        pltpu.VMEM((CH, 2 * F), jnp.float32),
        pltpu.VMEM((CH, 2 * F), jnp.float32),
        pltpu.SemaphoreType.DMA,
    ],
)(_sc_body)


def kernel(x, neighbor_indices, distancesq,
           Wd0, bd0, Wf0, bf0,
           Wd1, bd1, Wf1, bf1,
           Wd2, bd2, Wf2, bf2):
    zcol = jnp.zeros((1, D), jnp.float32)
    wdx = [Wd0[:, 0].reshape(1, D), Wd1[:D, 0].reshape(1, D),
           Wd2[:D, 0].reshape(1, D)]
    wdf = [zcol, Wd1[D:, 0].reshape(1, D), Wd2[D:, 0].reshape(1, D)]
    bd = [bd0.reshape(1, 1), bd1.reshape(1, 1), bd2.reshape(1, 1)]
    wfm = [Wf0, Wf1, Wf2]
    bfv = [bf0.reshape(1, F), bf1.reshape(1, F), bf2.reshape(1, F)]

    nidx_flat = jnp.pad(neighbor_indices, ((0, VP - V), (0, 0))).reshape(-1)

    outs = []
    featin = x
    d = distancesq
    for i in range(3):
        w, feat, d = _tc_stage(x, featin, d, wdx[i], wdf[i], bd[i],
                               wfm[i], bfv[i])
        feat_p = jnp.pad(feat, ((0, VP - V), (0, 0)))
        w_flat = jnp.pad(w, ((0, VP - V), (0, 0))).reshape(-1)
        out_p = _sc_knn(feat_p, nidx_flat, w_flat)
        out_i = out_p[:V]
        outs.append(out_i)
        featin = out_i
    return jnp.concatenate(outs + [x], axis=-1)


# speedup vs baseline: 1.7584x; 1.3130x over previous
import functools

import jax
import jax.numpy as jnp
from jax import lax
from jax.experimental import pallas as pl
from jax.experimental.pallas import tpu as pltpu
from jax.experimental.pallas import tpu_sc as plsc

V = 10000
D = 128
K = 32
F = 64

NC = 2
NS = 16
L = 16
NW = NC * NS
VP = 10240
RPW = VP // NW
CH = 4
GB = CH * K
NCHUNK = RPW // CH


def _tc_body(x_ref, f_ref, d_ref, wdx_ref, wdf_ref, bd_ref, wf_ref, bf_ref,
             w_ref, feat_ref, dn_ref):
    xb = x_ref[...]
    fb = f_ref[...]
    s = (jnp.sum(xb * wdx_ref[...], axis=1, keepdims=True)
         + jnp.sum(fb * wdf_ref[...], axis=1, keepdims=True)
         + bd_ref[0, 0])
    scale = 10.0 / (1.0 + jnp.exp(-s))
    dn = d_ref[...] * scale
    dn_ref[...] = dn
    w_ref[...] = jnp.exp(-10.0 * dn)
    ft = jnp.maximum(
        jnp.dot(fb, wf_ref[...], preferred_element_type=jnp.float32)
        + bf_ref[...], 0.0)
    feat_ref[...] = jnp.concatenate(
        [ft, jnp.zeros_like(ft)], axis=1)


def _tc_stage(x, featin, d, wdx, wdf, bd, wf, bf):
    bv = 1000
    grid = (V // bv,)
    return pl.pallas_call(
        _tc_body,
        grid=grid,
        in_specs=[
            pl.BlockSpec((bv, D), lambda i: (i, 0)),
            pl.BlockSpec((bv, D), lambda i: (i, 0)),
            pl.BlockSpec((bv, K), lambda i: (i, 0)),
            pl.BlockSpec((1, D), lambda i: (0, 0)),
            pl.BlockSpec((1, D), lambda i: (0, 0)),
            pl.BlockSpec((1, 1), lambda i: (0, 0)),
            pl.BlockSpec((D, F), lambda i: (0, 0)),
            pl.BlockSpec((1, F), lambda i: (0, 0)),
        ],
        out_specs=[
            pl.BlockSpec((bv, K), lambda i: (i, 0)),
            pl.BlockSpec((bv, 2 * F), lambda i: (i, 0)),
            pl.BlockSpec((bv, K), lambda i: (i, 0)),
        ],
        out_shape=[
            jax.ShapeDtypeStruct((V, K), jnp.float32),
            jax.ShapeDtypeStruct((V, 2 * F), jnp.float32),
            jax.ShapeDtypeStruct((V, K), jnp.float32),
        ],
    )(x, featin, d, wdx, wdf, bd, wf, bf)


def _sc_body(feat_hbm, nidxf_hbm, wflat_hbm, out_hbm,
             idx_all, w_all,
             rows0, rows1, own0, own1, out0, out1,
             semg0, semg1, semn0, semn1, semo0, semo1):
    wid = lax.axis_index("s") * NC + lax.axis_index("c")
    base = wid * RPW

    pltpu.sync_copy(nidxf_hbm.at[pl.ds(base * K, RPW * K)], idx_all)
    pltpu.sync_copy(wflat_hbm.at[pl.ds(base * K, RPW * K)], w_all)

    slots = ((rows0, own0, out0, semg0, semn0, semo0),
             (rows1, own1, out1, semg1, semn1, semo1))

    def fire(c, slot):
        rows_v, own_v, _, semg, semn, _ = slots[slot]
        pltpu.async_copy(feat_hbm.at[idx_all.at[pl.ds(c * GB, GB)]],
                         rows_v, semg)
        pltpu.async_copy(feat_hbm.at[pl.ds(base + c * CH, CH)], own_v, semn)

    fire(0, 0)
    fire(1, 1)

    def pair(i, carry):
        for slot in (0, 1):
            rows_v, own_v, out_v, semg, semn, semo = slots[slot]
            c = 2 * i + slot
            pltpu.make_async_copy(
                feat_hbm.at[idx_all.at[pl.ds(0, GB)]], rows_v, semg).wait()
            pltpu.make_async_copy(
                feat_hbm.at[pl.ds(0, CH)], own_v, semn).wait()

            @pl.when(i > 0)
            def _():
                pltpu.make_async_copy(
                    out_v, out_hbm.at[pl.ds(0, CH)], semo).wait()

            for dl in range(CH):
                row0 = dl * K
                accs = [jnp.zeros((L,), jnp.float32) for _ in range(F // L)]
                accm = [jnp.full((L,), -jnp.inf, jnp.float32)
                        for _ in range(F // L)]
                for kg in range(K // L):
                    w16 = w_all[pl.ds(c * GB + row0 + kg * L, L)]
                    for kl in range(L):
                        k = kg * L + kl
                        wv = jnp.full((L,), w16[kl])
                        for t in range(F // L):
                            nf = rows_v[row0 + k, pl.ds(t * L, L)]
                            wfv = wv * nf
                            accs[t] = accs[t] + wfv
                            accm[t] = jnp.maximum(accm[t], wfv)
                for t in range(F // L):
                    ov = own_v[dl, pl.ds(t * L, L)]
                    out_v[dl, pl.ds(t * L, L)] = accs[t] * (1.0 / K) - ov
                    out_v[dl, pl.ds(F + t * L, L)] = accm[t] - ov

            pltpu.async_copy(out_v, out_hbm.at[pl.ds(base + c * CH, CH)],
                             semo)

            @pl.when(c + 2 < NCHUNK)
            def _():
                fire(c + 2, slot)
        return carry

    lax.fori_loop(0, NCHUNK // 2, pair, 0)

    pltpu.make_async_copy(out0, out_hbm.at[pl.ds(0, CH)], semo0).wait()
    pltpu.make_async_copy(out1, out_hbm.at[pl.ds(0, CH)], semo1).wait()


_sc_knn = functools.partial(
    pl.kernel,
    out_type=jax.ShapeDtypeStruct((VP, 2 * F), jnp.float32),
    mesh=plsc.VectorSubcoreMesh(
        core_axis_name="c", subcore_axis_name="s",
        num_cores=NC, num_subcores=NS),
    scratch_types=[
        pltpu.VMEM((RPW * K,), jnp.int32),
        pltpu.VMEM((RPW * K,), jnp.float32),
        pltpu.VMEM((GB, 2 * F), jnp.float32),
        pltpu.VMEM((GB, 2 * F), jnp.float32),
        pltpu.VMEM((CH, 2 * F), jnp.float32),
        pltpu.VMEM((CH, 2 * F), jnp.float32),
        pltpu.VMEM((CH, 2 * F), jnp.float32),
        pltpu.VMEM((CH, 2 * F), jnp.float32),
        pltpu.SemaphoreType.DMA,
        pltpu.SemaphoreType.DMA,
        pltpu.SemaphoreType.DMA,
        pltpu.SemaphoreType.DMA,
        pltpu.SemaphoreType.DMA,
        pltpu.SemaphoreType.DMA,
    ],
)(_sc_body)


def kernel(x, neighbor_indices, distancesq,
           Wd0, bd0, Wf0, bf0,
           Wd1, bd1, Wf1, bf1,
           Wd2, bd2, Wf2, bf2):
    zcol = jnp.zeros((1, D), jnp.float32)
    wdx = [Wd0[:, 0].reshape(1, D), Wd1[:D, 0].reshape(1, D),
           Wd2[:D, 0].reshape(1, D)]
    wdf = [zcol, Wd1[D:, 0].reshape(1, D), Wd2[D:, 0].reshape(1, D)]
    bd = [bd0.reshape(1, 1), bd1.reshape(1, 1), bd2.reshape(1, 1)]
    wfm = [Wf0, Wf1, Wf2]
    bfv = [bf0.reshape(1, F), bf1.reshape(1, F), bf2.reshape(1, F)]

    nidx_flat = jnp.pad(neighbor_indices, ((0, VP - V), (0, 0))).reshape(-1)

    outs = []
    featin = x
    d = distancesq
    for i in range(3):
        w, feat, d = _tc_stage(x, featin, d, wdx[i], wdf[i], bd[i],
                               wfm[i], bfv[i])
        feat_p = jnp.pad(feat, ((0, VP - V), (0, 0)))
        w_flat = jnp.pad(w, ((0, VP - V), (0, 0))).reshape(-1)
        out_p = _sc_knn(feat_p, nidx_flat, w_flat)
        out_i = out_p[:V]
        outs.append(out_i)
        featin = out_i
    return jnp.concatenate(outs + [x], axis=-1)


# speedup vs baseline: 9.5466x; 5.4292x over previous
import functools

import jax
import jax.numpy as jnp
from jax import lax
from jax.experimental import pallas as pl
from jax.experimental.pallas import tpu as pltpu
from jax.experimental.pallas import tpu_sc as plsc

V = 10000
D = 128
K = 32
F = 64

NC = 2
NS = 16
L = 16
NW = NC * NS
VP = 10240
RPW = VP // NW
CH = 4
GB = CH * K
NCHUNK = RPW // CH


def _tc_body(x_ref, f_ref, d_ref, wdx_ref, wdf_ref, bd_ref, wf_ref, bf_ref,
             w_ref, feat_ref, dn_ref):
    xb = x_ref[...]
    fb = f_ref[...]
    s = (jnp.sum(xb * wdx_ref[...], axis=1, keepdims=True)
         + jnp.sum(fb * wdf_ref[...], axis=1, keepdims=True)
         + bd_ref[0, 0])
    scale = 10.0 / (1.0 + jnp.exp(-s))
    dn = d_ref[...] * scale
    dn_ref[...] = dn
    w_ref[...] = jnp.exp(-10.0 * dn)
    feat_ref[...] = jnp.maximum(
        jnp.dot(fb, wf_ref[...], preferred_element_type=jnp.float32)
        + bf_ref[...], 0.0)


def _tc_stage(x, featin, d, wdx, wdf, bd, wf, bf):
    bv = 1000
    grid = (V // bv,)
    return pl.pallas_call(
        _tc_body,
        grid=grid,
        in_specs=[
            pl.BlockSpec((bv, D), lambda i: (i, 0)),
            pl.BlockSpec((bv, D), lambda i: (i, 0)),
            pl.BlockSpec((bv, K), lambda i: (i, 0)),
            pl.BlockSpec((1, D), lambda i: (0, 0)),
            pl.BlockSpec((1, D), lambda i: (0, 0)),
            pl.BlockSpec((1, 1), lambda i: (0, 0)),
            pl.BlockSpec((D, F), lambda i: (0, 0)),
            pl.BlockSpec((1, F), lambda i: (0, 0)),
        ],
        out_specs=[
            pl.BlockSpec((bv, K), lambda i: (i, 0)),
            pl.BlockSpec((bv, F), lambda i: (i, 0)),
            pl.BlockSpec((bv, K), lambda i: (i, 0)),
        ],
        out_shape=[
            jax.ShapeDtypeStruct((V, K), jnp.float32),
            jax.ShapeDtypeStruct((V, F), jnp.float32),
            jax.ShapeDtypeStruct((V, K), jnp.float32),
        ],
    )(x, featin, d, wdx, wdf, bd, wf, bf)


def _sc_body(feat_hbm, nidxf_hbm, wflat_hbm, out_hbm,
             idx_all, w_all, tab_sh,
             rows0, rows1, own0, own1, out0, out1,
             semg0, semg1, semn0, semn1, semo0, semo1):
    sid = lax.axis_index("s")
    wid = sid * NC + lax.axis_index("c")
    base = wid * RPW

    stripe = VP // NS
    pltpu.sync_copy(feat_hbm.at[pl.ds(sid * stripe, stripe)],
                    tab_sh.at[pl.ds(sid * stripe, stripe)])
    pltpu.sync_copy(nidxf_hbm.at[pl.ds(base * K, RPW * K)], idx_all)
    pltpu.sync_copy(wflat_hbm.at[pl.ds(base * K, RPW * K)], w_all)
    plsc.subcore_barrier()

    slots = ((rows0, own0, out0, semg0, semn0, semo0),
             (rows1, own1, out1, semg1, semn1, semo1))

    def fire(c, slot):
        rows_v, own_v, _, semg, semn, _ = slots[slot]
        pltpu.async_copy(tab_sh.at[idx_all.at[pl.ds(c * GB, GB)]],
                         rows_v, semg)
        pltpu.async_copy(tab_sh.at[pl.ds(base + c * CH, CH)], own_v, semn)

    fire(0, 0)
    fire(1, 1)

    def pair(i, carry):
        for slot in (0, 1):
            rows_v, own_v, out_v, semg, semn, semo = slots[slot]
            c = 2 * i + slot
            pltpu.make_async_copy(
                tab_sh.at[idx_all.at[pl.ds(0, GB)]], rows_v, semg).wait()
            pltpu.make_async_copy(
                tab_sh.at[pl.ds(0, CH)], own_v, semn).wait()

            @pl.when(i > 0)
            def _():
                pltpu.make_async_copy(
                    out_v, out_hbm.at[pl.ds(0, CH)], semo).wait()

            for dl in range(CH):
                row0 = dl * K
                accs = [jnp.zeros((L,), jnp.float32) for _ in range(F // L)]
                accm = [jnp.full((L,), -jnp.inf, jnp.float32)
                        for _ in range(F // L)]
                for kg in range(K // L):
                    w16 = w_all[pl.ds(c * GB + row0 + kg * L, L)]
                    for kl in range(L):
                        k = kg * L + kl
                        wv = jnp.full((L,), w16[kl])
                        for t in range(F // L):
                            nf = rows_v[row0 + k, pl.ds(t * L, L)]
                            wfv = wv * nf
                            accs[t] = accs[t] + wfv
                            accm[t] = jnp.maximum(accm[t], wfv)
                for t in range(F // L):
                    ov = own_v[dl, pl.ds(t * L, L)]
                    out_v[dl, pl.ds(t * L, L)] = accs[t] * (1.0 / K) - ov
                    out_v[dl, pl.ds(F + t * L, L)] = accm[t] - ov

            pltpu.async_copy(out_v, out_hbm.at[pl.ds(base + c * CH, CH)],
                             semo)

            @pl.when(c + 2 < NCHUNK)
            def _():
                fire(c + 2, slot)
        return carry

    lax.fori_loop(0, NCHUNK // 2, pair, 0)

    pltpu.make_async_copy(out0, out_hbm.at[pl.ds(0, CH)], semo0).wait()
    pltpu.make_async_copy(out1, out_hbm.at[pl.ds(0, CH)], semo1).wait()


_sc_knn = functools.partial(
    pl.kernel,
    out_type=jax.ShapeDtypeStruct((VP, 2 * F), jnp.float32),
    mesh=plsc.VectorSubcoreMesh(
        core_axis_name="c", subcore_axis_name="s",
        num_cores=NC, num_subcores=NS),
    scratch_types=[
        pltpu.VMEM((RPW * K,), jnp.int32),
        pltpu.VMEM((RPW * K,), jnp.float32),
        pltpu.VMEM_SHARED((VP, F), jnp.float32),
        pltpu.VMEM((GB, F), jnp.float32),
        pltpu.VMEM((GB, F), jnp.float32),
        pltpu.VMEM((CH, F), jnp.float32),
        pltpu.VMEM((CH, F), jnp.float32),
        pltpu.VMEM((CH, 2 * F), jnp.float32),
        pltpu.VMEM((CH, 2 * F), jnp.float32),
        pltpu.SemaphoreType.DMA,
        pltpu.SemaphoreType.DMA,
        pltpu.SemaphoreType.DMA,
        pltpu.SemaphoreType.DMA,
        pltpu.SemaphoreType.DMA,
        pltpu.SemaphoreType.DMA,
    ],
)(_sc_body)


def kernel(x, neighbor_indices, distancesq,
           Wd0, bd0, Wf0, bf0,
           Wd1, bd1, Wf1, bf1,
           Wd2, bd2, Wf2, bf2):
    zcol = jnp.zeros((1, D), jnp.float32)
    wdx = [Wd0[:, 0].reshape(1, D), Wd1[:D, 0].reshape(1, D),
           Wd2[:D, 0].reshape(1, D)]
    wdf = [zcol, Wd1[D:, 0].reshape(1, D), Wd2[D:, 0].reshape(1, D)]
    bd = [bd0.reshape(1, 1), bd1.reshape(1, 1), bd2.reshape(1, 1)]
    wfm = [Wf0, Wf1, Wf2]
    bfv = [bf0.reshape(1, F), bf1.reshape(1, F), bf2.reshape(1, F)]

    nidx_flat = jnp.pad(neighbor_indices, ((0, VP - V), (0, 0))).reshape(-1)

    outs = []
    featin = x
    d = distancesq
    for i in range(3):
        w, feat, d = _tc_stage(x, featin, d, wdx[i], wdf[i], bd[i],
                               wfm[i], bfv[i])
        feat_p = jnp.pad(feat, ((0, VP - V), (0, 0)))
        w_flat = jnp.pad(w, ((0, VP - V), (0, 0))).reshape(-1)
        out_p = _sc_knn(feat_p, nidx_flat, w_flat)
        out_i = out_p[:V]
        outs.append(out_i)
        featin = out_i
    return jnp.concatenate(outs + [x], axis=-1)
